# baseline (speedup 1.0000x reference)
import functools

import jax
import jax.numpy as jnp
from jax import lax
from jax.experimental import pallas as pl
from jax.experimental.pallas import tpu as pltpu
from jax.experimental.pallas import tpu_sc as plsc

N = 10000
T = 128
H = 512
L = 128
E = 320000

NC = 2
NS = 16
NW = NC * NS
EPW = E // NW
CHUNK = 80
SECT = 16
EPWP = 10240
NSEC = EPWP // (SECT * CHUNK)
NPAD = EPWP - EPW
NBUF = 4
RPT = 624
TAIL = N - NS * RPT


def _seg_sum_sc(xs, src3, dst3, zeros_rpt):
    nslab = len(xs)
    mesh = plsc.VectorSubcoreMesh(core_axis_name="c", subcore_axis_name="s")

    @functools.partial(
        pl.kernel,
        mesh=mesh,
        out_type=jax.ShapeDtypeStruct((NC, nslab, N, L), jnp.float32),
        scratch_types=[
            pltpu.VMEM((2, SECT, CHUNK), jnp.int32),
            pltpu.VMEM((2, SECT, CHUNK), jnp.int32),
            pltpu.VMEM((NBUF, CHUNK, L), jnp.float32),
            pltpu.VMEM_SHARED((N + 8, L), jnp.float32),
            pltpu.SemaphoreType.DMA,
            pltpu.SemaphoreType.DMA,
            pltpu.SemaphoreType.DMA,
        ],
    )
    def k(*refs):
        x_hbms = refs[:nslab]
        (src_hbm, dst_hbm, z_hbm, out_hbm, src_v, dst_v, rows_v, acc,
         gsem, ssem, isem) = refs[nslab:]
        c = lax.axis_index("c")
        s = lax.axis_index("s")
        w = c * NS + s
        src_w = src_hbm.at[w]
        dst_w = dst_hbm.at[w]

        def wait_gather():
            pltpu.make_async_copy(x_hbms[0].at[src_v.at[0].at[0]],
                                  rows_v.at[0], gsem).wait()

        def drain_scatter():
            pltpu.make_async_copy(rows_v.at[0], acc.at[dst_v.at[0].at[0]],
                                  ssem).wait()

        def drain_stage():
            pltpu.make_async_copy(src_w.at[0], src_v.at[0], isem).wait()

        for q, x_hbm in enumerate(x_hbms):
            pltpu.sync_copy(src_w.at[0], src_v.at[0])
            pltpu.sync_copy(dst_w.at[0], dst_v.at[0])
            pltpu.async_copy(x_hbm.at[src_v.at[0].at[0]], rows_v.at[0], gsem)
            pltpu.async_copy(x_hbm.at[src_v.at[0].at[1]], rows_v.at[1], gsem)
            pltpu.sync_copy(z_hbm, acc.at[pl.ds(s * RPT, RPT)])

            @pl.when(s == NS - 1)
            def _():
                pltpu.sync_copy(z_hbm.at[pl.ds(0, TAIL)],
                                acc.at[pl.ds(NS * RPT, TAIL)])

            plsc.subcore_barrier()

            for sec in range(NSEC):
                p = sec & 1
                ssec = src_v.at[p]
                dsec = dst_v.at[p]

                for g in (0, 1):
                    wait_gather()
                    if sec >= 1:
                        drain_scatter()
                    pltpu.async_copy(rows_v.at[g], acc.at[dsec.at[g]],
                                     ssem, add=True)
                    pltpu.async_copy(x_hbm.at[ssec.at[g + 2]],
                                     rows_v.at[g + 2], gsem)

                if sec + 1 < NSEC:
                    pltpu.async_copy(src_w.at[sec + 1], src_v.at[1 - p], isem)
                    pltpu.async_copy(dst_w.at[sec + 1], dst_v.at[1 - p], isem)

                def step(g, carry):
                    b = rows_v.at[lax.rem(g, NBUF)]
                    wait_gather()
                    drain_scatter()
                    pltpu.async_copy(b, acc.at[dsec.at[g]], ssem, add=True)
                    pltpu.async_copy(x_hbm.at[ssec.at[g + 2]],
                                     rows_v.at[lax.rem(g + 2, NBUF)], gsem)
                    return carry

                lax.fori_loop(2, SECT - 2, step, 0)

                for g in (SECT - 2, SECT - 1):
                    b = rows_v.at[g % NBUF]
                    wait_gather()
                    drain_scatter()
                    pltpu.async_copy(b, acc.at[dsec.at[g]], ssem, add=True)
                    if sec + 1 < NSEC:
                        if g == SECT - 2:
                            drain_stage()
                            drain_stage()
                        pltpu.async_copy(
                            x_hbm.at[src_v.at[1 - p].at[g - (SECT - 2)]],
                            rows_v.at[(g + 2) % NBUF], gsem)

            drain_scatter()
            drain_scatter()
            plsc.subcore_barrier()
            out_q = out_hbm.at[c].at[q]
            pltpu.sync_copy(acc.at[pl.ds(s * RPT, RPT)],
                            out_q.at[pl.ds(s * RPT, RPT)])

            @pl.when(s == NS - 1)
            def _():
                pltpu.sync_copy(acc.at[pl.ds(NS * RPT, TAIL)],
                                out_q.at[pl.ds(NS * RPT, TAIL)])

    return k(*xs, src3, dst3, zeros_rpt)


BLK = 1000


def _bdot(a, b):
    return jnp.dot(a.astype(jnp.bfloat16), b.astype(jnp.bfloat16),
                   preferred_element_type=jnp.float32)


def _mlp1(x, a0, a1, W1, b1, W2, b2):
    def body(x_r, a0_r, a1_r, W1_r, b1_r, W2_r, b2_r, y0_r, y1_r, y2_r, y3_r):
        h = x_r[...] + a0_r[...] + a1_r[...]
        h = jnp.maximum(_bdot(h, W1_r[...]) + b1_r[...], 0.0)
        x1 = jnp.maximum(_bdot(h, W2_r[...]) + b2_r[...], 0.0)
        for q, y_r in enumerate((y0_r, y1_r, y2_r, y3_r)):
            y_r[...] = x1[:, q * L:(q + 1) * L]

    slab = jax.ShapeDtypeStruct((N, L), jnp.float32)
    return pl.pallas_call(
        body,
        grid=(N // BLK,),
        in_specs=[
            pl.BlockSpec((BLK, T), lambda i: (i, 0)),
            pl.BlockSpec((BLK, T), lambda i: (i, 0)),
            pl.BlockSpec((BLK, T), lambda i: (i, 0)),
            pl.BlockSpec((T, H), lambda i: (0, 0)),
            pl.BlockSpec((1, H), lambda i: (0, 0)),
            pl.BlockSpec((H, H), lambda i: (0, 0)),
            pl.BlockSpec((1, H), lambda i: (0, 0)),
        ],
        out_specs=[pl.BlockSpec((BLK, L), lambda i: (i, 0))] * 4,
        out_shape=[slab] * 4,
    )(x, a0, a1, W1, b1, W2, b2)


def _head(x1s, aggs, b1, W1, W2, b2, wd, bd):
    def body(*refs):
        (x0_r, x1_r, x2_r, x3_r,
         a00_r, a01_r, a02_r, a03_r, a10_r, a11_r, a12_r, a13_r,
         b1_r, W1_r, W2_r, b2_r, wd_r, bd_r, o_r) = refs
        i = pl.program_id(0)
        xs = (x0_r, x1_r, x2_r, x3_r)
        c0 = (a00_r, a01_r, a02_r, a03_r)
        c1 = (a10_r, a11_r, a12_r, a13_r)
        s = jnp.concatenate(
            [xs[q][...] + c0[q][...] + c1[q][...] for q in range(4)], axis=1)
        h = jnp.maximum(_bdot(s, W1_r[...]) + b1_r[...], 0.0)
        x2 = _bdot(h, W2_r[...]) + b2_r[...]
        part = jnp.sum(x2 * wd_r[...])

        @pl.when(i == 0)
        def _():
            o_r[...] = bd_r[...]

        o_r[...] = o_r[...] + part

        @pl.when(i == pl.num_programs(0) - 1)
        def _():
            o_r[...] = jax.nn.sigmoid(o_r[...])

    blk_l = pl.BlockSpec((BLK, L), lambda i: (i, 0))
    return pl.pallas_call(
        body,
        grid=(N // BLK,),
        in_specs=(
            [blk_l] * 12 + [
                pl.BlockSpec((1, L), lambda i: (0, 0)),
                pl.BlockSpec((H, L), lambda i: (0, 0)),
                pl.BlockSpec((L, L), lambda i: (0, 0)),
                pl.BlockSpec((1, L), lambda i: (0, 0)),
                blk_l,
                pl.BlockSpec((1, 1), lambda i: (0, 0)),
            ]
        ),
        out_specs=pl.BlockSpec((1, 1), lambda i: (0, 0)),
        out_shape=jax.ShapeDtypeStruct((1, 1), jnp.float32),
    )(*x1s, *aggs, b1, W1, W2, b2, wd, bd)


def kernel(eeg_nodes, eeg_idx, W1_1, b1_1, W2_1, b2_1, W1_2, b1_2, W2_2, b2_2, Wd, bd):
    src3 = jnp.concatenate(
        [eeg_idx[0].reshape(NW, EPW),
         jnp.zeros((NW, NPAD), jnp.int32)], axis=1
    ).reshape(NW, NSEC, SECT, CHUNK)
    dst3 = jnp.concatenate(
        [eeg_idx[1].reshape(NW, EPW),
         jnp.full((NW, NPAD), N, jnp.int32)], axis=1
    ).reshape(NW, NSEC, SECT, CHUNK)
    zeros_rpt = jnp.zeros((RPT, L), jnp.float32)

    agg1 = _seg_sum_sc((eeg_nodes,), src3, dst3, zeros_rpt)
    x1s = _mlp1(eeg_nodes, agg1[0, 0], agg1[1, 0],
                W1_1, b1_1.reshape(1, H), W2_1, b2_1.reshape(1, H))
    agg2 = _seg_sum_sc(tuple(x1s), src3, dst3, zeros_rpt)
    aggs = [agg2[c, q] for c in range(NC) for q in range(4)]
    out = _head(x1s, aggs, b1_2.reshape(1, L), W1_2,
                W2_2, b2_2.reshape(1, L), Wd.reshape(N, L), bd.reshape(1, 1))
    return out


# speedup vs baseline: 2.7267x; 2.7267x over previous
import functools

import jax
import jax.numpy as jnp
from jax import lax
from jax.experimental import pallas as pl
from jax.experimental.pallas import tpu as pltpu
from jax.experimental.pallas import tpu_sc as plsc

N = 10000
T = 128
H = 512
L = 128
E = 320000

NC = 2
NS = 16
NW = NC * NS
EPW = E // NW
CHUNK = 125
NCHUNK = EPW // CHUNK
SECT = 16
NSEC = NCHUNK // SECT
RPT = 624
TAIL = N - NS * RPT


def _seg_sum_sc(xs, src3, dst3, zeros_rpt):
    nslab = len(xs)
    mesh = plsc.VectorSubcoreMesh(core_axis_name="c", subcore_axis_name="s")

    @functools.partial(
        pl.kernel,
        mesh=mesh,
        out_type=jax.ShapeDtypeStruct((NC, nslab, N, L), jnp.float32),
        scratch_types=[
            pltpu.VMEM((2, SECT, CHUNK), jnp.int32),
            pltpu.VMEM((2, SECT, CHUNK), jnp.int32),
            pltpu.VMEM((NBUF, CHUNK, L), jnp.float32),
            pltpu.VMEM_SHARED((N + 8, L), jnp.float32),
            pltpu.SemaphoreType.DMA,
            pltpu.SemaphoreType.DMA,
            pltpu.SemaphoreType.DMA,
        ],
    )
    def k(*refs):
        x_hbms = refs[:nslab]
        (src_hbm, dst_hbm, z_hbm, out_hbm, src_v, dst_v, rows_v, acc,
         gsem, ssem, isem) = refs[nslab:]
        c = lax.axis_index("c")
        s = lax.axis_index("s")
        w = c * NS + s
        src_w = src_hbm.at[w]
        dst_w = dst_hbm.at[w]

        def wait_gather():
            pltpu.make_async_copy(x_hbms[0].at[src_v.at[0].at[0]],
                                  rows_v.at[0], gsem).wait()

        def drain_scatter():
            pltpu.make_async_copy(rows_v.at[0], acc.at[dst_v.at[0].at[0]],
                                  ssem).wait()

        def drain_stage():
            pltpu.make_async_copy(src_w.at[0], src_v.at[0], isem).wait()

        for q, x_hbm in enumerate(x_hbms):
            pltpu.sync_copy(src_w.at[0], src_v.at[0])
            pltpu.sync_copy(dst_w.at[0], dst_v.at[0])
            pltpu.async_copy(x_hbm.at[src_v.at[0].at[0]], rows_v.at[0], gsem)
            pltpu.async_copy(x_hbm.at[src_v.at[0].at[1]], rows_v.at[1], gsem)
            pltpu.sync_copy(z_hbm, acc.at[pl.ds(s * RPT, RPT)])

            @pl.when(s == NS - 1)
            def _():
                pltpu.sync_copy(z_hbm.at[pl.ds(0, TAIL)],
                                acc.at[pl.ds(NS * RPT, TAIL)])

            plsc.subcore_barrier()

            for sec in range(NSEC):
                p = sec & 1
                ssec = src_v.at[p]
                dsec = dst_v.at[p]

                for g in (0, 1):
                    wait_gather()
                    if sec >= 1:
                        drain_scatter()
                    pltpu.async_copy(rows_v.at[g], acc.at[dsec.at[g]],
                                     ssem, add=True)
                    pltpu.async_copy(x_hbm.at[ssec.at[g + 2]],
                                     rows_v.at[g + 2], gsem)

                if sec + 1 < NSEC:
                    pltpu.async_copy(src_w.at[sec + 1], src_v.at[1 - p], isem)
                    pltpu.async_copy(dst_w.at[sec + 1], dst_v.at[1 - p], isem)

                def step(g, carry):
                    b = rows_v.at[lax.rem(g, NBUF)]
                    wait_gather()
                    drain_scatter()
                    pltpu.async_copy(b, acc.at[dsec.at[g]], ssem, add=True)
                    pltpu.async_copy(x_hbm.at[ssec.at[g + 2]],
                                     rows_v.at[lax.rem(g + 2, NBUF)], gsem)
                    return carry

                lax.fori_loop(2, SECT - 2, step, 0)

                for g in (SECT - 2, SECT - 1):
                    b = rows_v.at[g % NBUF]
                    wait_gather()
                    drain_scatter()
                    pltpu.async_copy(b, acc.at[dsec.at[g]], ssem, add=True)
                    if sec + 1 < NSEC:
                        if g == SECT - 2:
                            drain_stage()
                            drain_stage()
                        pltpu.async_copy(
                            x_hbm.at[src_v.at[1 - p].at[g - (SECT - 2)]],
                            rows_v.at[(g + 2) % NBUF], gsem)

            drain_scatter()
            drain_scatter()
            plsc.subcore_barrier()
            out_q = out_hbm.at[c].at[q]
            pltpu.sync_copy(acc.at[pl.ds(s * RPT, RPT)],
                            out_q.at[pl.ds(s * RPT, RPT)])

            @pl.when(s == NS - 1)
            def _():
                pltpu.sync_copy(acc.at[pl.ds(NS * RPT, TAIL)],
                                out_q.at[pl.ds(NS * RPT, TAIL)])

    return k(*xs, src3, dst3, zeros_rpt)


BLK = 1000


def _bdot(a, b):
    return jnp.dot(a.astype(jnp.bfloat16), b.astype(jnp.bfloat16),
                   preferred_element_type=jnp.float32)


def _mlp1(x, a0, a1, W1, b1, W2, b2):
    def body(x_r, a0_r, a1_r, W1_r, b1_r, W2_r, b2_r, y0_r, y1_r, y2_r, y3_r):
        h = x_r[...] + a0_r[...] + a1_r[...]
        h = jnp.maximum(_bdot(h, W1_r[...]) + b1_r[...], 0.0)
        x1 = jnp.maximum(_bdot(h, W2_r[...]) + b2_r[...], 0.0)
        for q, y_r in enumerate((y0_r, y1_r, y2_r, y3_r)):
            y_r[...] = x1[:, q * L:(q + 1) * L]

    slab = jax.ShapeDtypeStruct((N, L), jnp.float32)
    return pl.pallas_call(
        body,
        grid=(N // BLK,),
        in_specs=[
            pl.BlockSpec((BLK, T), lambda i: (i, 0)),
            pl.BlockSpec((BLK, T), lambda i: (i, 0)),
            pl.BlockSpec((BLK, T), lambda i: (i, 0)),
            pl.BlockSpec((T, H), lambda i: (0, 0)),
            pl.BlockSpec((1, H), lambda i: (0, 0)),
            pl.BlockSpec((H, H), lambda i: (0, 0)),
            pl.BlockSpec((1, H), lambda i: (0, 0)),
        ],
        out_specs=[pl.BlockSpec((BLK, L), lambda i: (i, 0))] * 4,
        out_shape=[slab] * 4,
    )(x, a0, a1, W1, b1, W2, b2)


def _head(x1s, aggs, b1, W1, W2, b2, wd, bd):
    def body(*refs):
        (x0_r, x1_r, x2_r, x3_r,
         a00_r, a01_r, a02_r, a03_r, a10_r, a11_r, a12_r, a13_r,
         b1_r, W1_r, W2_r, b2_r, wd_r, bd_r, o_r) = refs
        i = pl.program_id(0)
        xs = (x0_r, x1_r, x2_r, x3_r)
        c0 = (a00_r, a01_r, a02_r, a03_r)
        c1 = (a10_r, a11_r, a12_r, a13_r)
        s = jnp.concatenate(
            [xs[q][...] + c0[q][...] + c1[q][...] for q in range(4)], axis=1)
        h = jnp.maximum(_bdot(s, W1_r[...]) + b1_r[...], 0.0)
        x2 = _bdot(h, W2_r[...]) + b2_r[...]
        part = jnp.sum(x2 * wd_r[...])

        @pl.when(i == 0)
        def _():
            o_r[...] = bd_r[...]

        o_r[...] = o_r[...] + part

        @pl.when(i == pl.num_programs(0) - 1)
        def _():
            o_r[...] = jax.nn.sigmoid(o_r[...])

    blk_l = pl.BlockSpec((BLK, L), lambda i: (i, 0))
    return pl.pallas_call(
        body,
        grid=(N // BLK,),
        in_specs=(
            [blk_l] * 12 + [
                pl.BlockSpec((1, L), lambda i: (0, 0)),
                pl.BlockSpec((H, L), lambda i: (0, 0)),
                pl.BlockSpec((L, L), lambda i: (0, 0)),
                pl.BlockSpec((1, L), lambda i: (0, 0)),
                blk_l,
                pl.BlockSpec((1, 1), lambda i: (0, 0)),
            ]
        ),
        out_specs=pl.BlockSpec((1, 1), lambda i: (0, 0)),
        out_shape=jax.ShapeDtypeStruct((1, 1), jnp.float32),
    )(*x1s, *aggs, b1, W1, W2, b2, wd, bd)


def kernel(eeg_nodes, eeg_idx, W1_1, b1_1, W2_1, b2_1, W1_2, b1_2, W2_2, b2_2, Wd, bd):
    src3 = eeg_idx[0].reshape(NW, NCHUNK, CHUNK)
    dst3 = eeg_idx[1].reshape(NW, NCHUNK, CHUNK)
    zeros_rpt = jnp.zeros((RPT, L), jnp.float32)

    agg1 = _seg_sum_sc((eeg_nodes,), src3, dst3, zeros_rpt)
    x1s = _mlp1(eeg_nodes, agg1[0, 0], agg1[1, 0],
                W1_1, b1_1.reshape(1, H), W2_1, b2_1.reshape(1, H))
    agg2 = _seg_sum_sc(tuple(x1s), src3, dst3, zeros_rpt)
    aggs = [agg2[c, q] for c in range(NC) for q in range(4)]
    out = _head(x1s, aggs, b1_2.reshape(1, L), W1_2,
                W2_2, b2_2.reshape(1, L), Wd.reshape(N, L), bd.reshape(1, 1))
    return out
